# baseline (speedup 1.0000x reference)
import functools

import jax
import jax.numpy as jnp
from jax import lax
from jax.experimental import pallas as pl
from jax.experimental.pallas import tpu as pltpu
from jax.experimental.pallas import tpu_sc as plsc

N = 10000
E = 320000
IN_DIM = 128

L = 16
NCORES = 2
NTILES = 16
NW = NCORES * NTILES
NP = 10240
SLICE = NP // NTILES
EPT = E // NW
ECHUNK = 10240


def _matvec_body(f_hbm, w_ref, asrc_ref, adst_ref, xp_ref, att_ref,
                 f_vmem, sem):
    cp = pltpu.make_async_copy(f_hbm, f_vmem, sem)
    cp.start()
    cp.wait()
    xp_ref[:, :N] = jnp.dot(w_ref[...].T, f_vmem[...],
                            preferred_element_type=jnp.float32)
    xp_ref[:, N:] = jnp.zeros((1, NP - N), jnp.float32)
    att_ref[:, :L] = jnp.full((1, L), asrc_ref[0], jnp.float32)
    att_ref[:, L:] = jnp.full((1, L), adst_ref[0], jnp.float32)


def _xp_pallas(f, w, att_src, att_dst):
    return pl.pallas_call(
        _matvec_body,
        out_shape=[
            jax.ShapeDtypeStruct((1, NP), jnp.float32),
            jax.ShapeDtypeStruct((1, 2 * L), jnp.float32),
        ],
        in_specs=[
            pl.BlockSpec(memory_space=pltpu.MemorySpace.HBM),
            pl.BlockSpec(memory_space=pltpu.VMEM),
            pl.BlockSpec(memory_space=pltpu.SMEM),
            pl.BlockSpec(memory_space=pltpu.SMEM),
        ],
        scratch_shapes=[
            pltpu.VMEM((IN_DIM, N), jnp.float32),
            pltpu.SemaphoreType.DMA,
        ],
    )(f, w, att_src, att_dst)


_sc_mesh = plsc.VectorSubcoreMesh(core_axis_name="c", subcore_axis_name="s")


@functools.partial(
    pl.kernel,
    out_type=[
        jax.ShapeDtypeStruct((NCORES, NP), jnp.float32),
        jax.ShapeDtypeStruct((NCORES, NP), jnp.float32),
    ],
    mesh=_sc_mesh,
    compiler_params=pltpu.CompilerParams(needs_layout_passes=False),
    scratch_types=[
        pltpu.VMEM((NP,), jnp.float32),
        pltpu.VMEM((2, ECHUNK), jnp.int32),
        pltpu.VMEM((NP,), jnp.float32),
        pltpu.VMEM((NP,), jnp.float32),
        pltpu.VMEM((2 * L,), jnp.float32),
        pltpu.VMEM((NTILES, SLICE), jnp.float32),
        pltpu.VMEM_SHARED((NTILES, NP), jnp.float32),
        pltpu.VMEM_SHARED((NTILES, NP), jnp.float32),
        pltpu.SemaphoreType.DMA,
        pltpu.SemaphoreType.DMA,
        pltpu.SemaphoreType.DMA,
    ],
)
def _edge_kernel(xp_hbm, edge_hbm, att_hbm, num_out, den_out,
                 xp_v, ev_v, num_v, den_v, att_v, red_v,
                 num_sh, den_sh, sem0, sem1, sem2):
    cid = lax.axis_index("c")
    sid = lax.axis_index("s")
    wid = sid * NCORES + cid

    base = wid * EPT
    abase = jnp.minimum(base - lax.rem(base, 128), E - ECHUNK)
    abase = pl.multiple_of(abase, 128)
    o = base - abase

    cp_e = pltpu.async_copy(edge_hbm.at[:, pl.ds(abase, ECHUNK)], ev_v, sem0)
    cp_x = pltpu.async_copy(xp_hbm.at[0], xp_v, sem1)
    cp_a = pltpu.async_copy(att_hbm.at[0], att_v, sem2)

    zero = jnp.zeros((L,), jnp.float32)

    @plsc.parallel_loop(0, NP // L, unroll=4)
    def _(j):
        num_v[pl.ds(j * L, L)] = zero
        den_v[pl.ds(j * L, L)] = zero

    cp_e.wait()
    cp_x.wait()
    cp_a.wait()

    a_s = att_v[pl.ds(0, L)]
    a_d = att_v[pl.ds(L, L)]
    a_sum = a_s + a_d

    @plsc.parallel_loop(0, EPT // L, unroll=5)
    def _(i):
        sv = ev_v[0, pl.ds(o + i * L, L)]
        dv = ev_v[1, pl.ds(o + i * L, L)]
        xs = plsc.load_gather(xp_v, [sv])
        xd = plsc.load_gather(xp_v, [dv])
        t = a_s * xs + a_d * xd
        alpha = jnp.where(t < 0.0, t * 0.2, t)
        c = a_sum * xd
        c = jnp.where(c < 0.0, c * 0.2, c)
        ex = jnp.exp(alpha - c)
        plsc.addupdate_scatter(den_v, [dv], ex)
        plsc.addupdate_scatter(num_v, [dv], ex * xs)

    pltpu.sync_copy(num_v, num_sh.at[sid])
    pltpu.sync_copy(den_v, den_sh.at[sid])
    plsc.subcore_barrier()

    nbase = sid * SLICE

    pltpu.sync_copy(num_sh.at[:, pl.ds(nbase, SLICE)], red_v)

    @plsc.parallel_loop(0, SLICE // L, unroll=4)
    def _(j):
        acc = red_v[0, pl.ds(j * L, L)]
        for k in range(1, NTILES):
            acc = acc + red_v[k, pl.ds(j * L, L)]
        num_v[pl.ds(j * L, L)] = acc

    pltpu.sync_copy(num_v.at[pl.ds(0, SLICE)], num_out.at[cid, pl.ds(nbase, SLICE)])

    pltpu.sync_copy(den_sh.at[:, pl.ds(nbase, SLICE)], red_v)

    @plsc.parallel_loop(0, SLICE // L, unroll=4)
    def _(j):
        acc = red_v[0, pl.ds(j * L, L)]
        for k in range(1, NTILES):
            acc = acc + red_v[k, pl.ds(j * L, L)]
        den_v[pl.ds(j * L, L)] = acc

    pltpu.sync_copy(den_v.at[pl.ds(0, SLICE)], den_out.at[cid, pl.ds(nbase, SLICE)])


def _combine_body(xp_ref, n_ref, d_ref, b_ref, o_ref):
    num = xp_ref[:, :N] + n_ref[0:1, :N] + n_ref[1:2, :N]
    den = 1.0 + d_ref[0:1, :N] + d_ref[1:2, :N] + 1e-16
    o_ref[...] = (num / den + b_ref[0])[0]


def _combine_pallas(xp_pad, num_p, den_p, bias):
    return pl.pallas_call(
        _combine_body,
        out_shape=jax.ShapeDtypeStruct((N,), jnp.float32),
        in_specs=[
            pl.BlockSpec(memory_space=pltpu.VMEM),
            pl.BlockSpec(memory_space=pltpu.VMEM),
            pl.BlockSpec(memory_space=pltpu.VMEM),
            pl.BlockSpec(memory_space=pltpu.SMEM),
        ],
    )(xp_pad, num_p, den_p, bias)


def kernel(F, edge_index, W, att_src, att_dst, bias):
    xp2, att = _xp_pallas(F, W, att_src, att_dst)
    num_p, den_p = _edge_kernel(xp2, edge_index, att)
    return _combine_pallas(xp2, num_p, den_p, bias)


# speedup vs baseline: 1.0233x; 1.0233x over previous
import functools

import jax
import jax.numpy as jnp
from jax import lax
from jax.experimental import pallas as pl
from jax.experimental.pallas import tpu as pltpu
from jax.experimental.pallas import tpu_sc as plsc

N = 10000
E = 320000
IN_DIM = 128

L = 16
NCORES = 2
NTILES = 16
NW = NCORES * NTILES
NP = 10240
SLICE = NP // NTILES
EPT = E // NW
ECHUNK = 10240


MV_BLK = 2048
MV_GRID = (N + MV_BLK - 1) // MV_BLK


def _matvec_body(f_ref, w_ref, xp_ref):
    xp_ref[...] = jnp.dot(w_ref[...].T, f_ref[...],
                          preferred_element_type=jnp.float32)


def _xp_pallas(f, w):
    return pl.pallas_call(
        _matvec_body,
        grid=(MV_GRID,),
        out_shape=jax.ShapeDtypeStruct((1, N), jnp.float32),
        in_specs=[
            pl.BlockSpec((IN_DIM, MV_BLK), lambda i: (0, i)),
            pl.BlockSpec((IN_DIM, 1), lambda i: (0, 0)),
        ],
        out_specs=pl.BlockSpec((1, MV_BLK), lambda i: (0, i)),
    )(f, w)


_sc_mesh = plsc.VectorSubcoreMesh(core_axis_name="c", subcore_axis_name="s")


@functools.partial(
    pl.kernel,
    out_type=[
        jax.ShapeDtypeStruct((NCORES, NP), jnp.float32),
        jax.ShapeDtypeStruct((NCORES, NP), jnp.float32),
    ],
    mesh=_sc_mesh,
    compiler_params=pltpu.CompilerParams(needs_layout_passes=False),
    scratch_types=[
        pltpu.VMEM((N,), jnp.float32),
        pltpu.VMEM((2, ECHUNK), jnp.int32),
        pltpu.VMEM((NP,), jnp.float32),
        pltpu.VMEM((NP,), jnp.float32),
        pltpu.VMEM((1,), jnp.float32),
        pltpu.VMEM((1,), jnp.float32),
        pltpu.VMEM((NTILES, SLICE), jnp.float32),
        pltpu.VMEM_SHARED((NTILES, NP), jnp.float32),
        pltpu.VMEM_SHARED((NTILES, NP), jnp.float32),
        pltpu.SemaphoreType.DMA,
        pltpu.SemaphoreType.DMA,
        pltpu.SemaphoreType.DMA,
        pltpu.SemaphoreType.DMA,
    ],
)
def _edge_kernel(xp_hbm, edge_hbm, asrc_hbm, adst_hbm, num_out, den_out,
                 xp_v, ev_v, num_v, den_v, as_v, ad_v, red_v,
                 num_sh, den_sh, sem0, sem1, sem2, sem3):
    cid = lax.axis_index("c")
    sid = lax.axis_index("s")
    wid = sid * NCORES + cid

    base = wid * EPT
    abase = jnp.minimum(base - lax.rem(base, 128), E - ECHUNK)
    abase = pl.multiple_of(abase, 128)
    o = base - abase

    cp_e = pltpu.async_copy(edge_hbm.at[:, pl.ds(abase, ECHUNK)], ev_v, sem0)
    cp_x = pltpu.async_copy(xp_hbm.at[0], xp_v, sem1)
    cp_s = pltpu.async_copy(asrc_hbm, as_v, sem2)
    cp_d = pltpu.async_copy(adst_hbm, ad_v, sem3)

    zero = jnp.zeros((L,), jnp.float32)

    @plsc.parallel_loop(0, NP // L, unroll=4)
    def _(j):
        num_v[pl.ds(j * L, L)] = zero
        den_v[pl.ds(j * L, L)] = zero

    cp_e.wait()
    cp_x.wait()
    cp_s.wait()
    cp_d.wait()

    zidx = jnp.zeros((L,), jnp.int32)
    a_s = plsc.load_gather(as_v, [zidx])
    a_d = plsc.load_gather(ad_v, [zidx])
    a_sum = a_s + a_d

    @plsc.parallel_loop(0, EPT // L, unroll=5)
    def _(i):
        sv = ev_v[0, pl.ds(o + i * L, L)]
        dv = ev_v[1, pl.ds(o + i * L, L)]
        xs = plsc.load_gather(xp_v, [sv])
        xd = plsc.load_gather(xp_v, [dv])
        t = a_s * xs + a_d * xd
        alpha = jnp.where(t < 0.0, t * 0.2, t)
        c = a_sum * xd
        c = jnp.where(c < 0.0, c * 0.2, c)
        ex = jnp.exp(alpha - c)
        plsc.addupdate_scatter(den_v, [dv], ex)
        plsc.addupdate_scatter(num_v, [dv], ex * xs)

    pltpu.sync_copy(num_v, num_sh.at[sid])
    pltpu.sync_copy(den_v, den_sh.at[sid])
    plsc.subcore_barrier()

    nbase = sid * SLICE

    pltpu.sync_copy(num_sh.at[:, pl.ds(nbase, SLICE)], red_v)

    @plsc.parallel_loop(0, SLICE // L, unroll=4)
    def _(j):
        acc = red_v[0, pl.ds(j * L, L)]
        for k in range(1, NTILES):
            acc = acc + red_v[k, pl.ds(j * L, L)]
        num_v[pl.ds(j * L, L)] = acc

    pltpu.sync_copy(num_v.at[pl.ds(0, SLICE)], num_out.at[cid, pl.ds(nbase, SLICE)])

    pltpu.sync_copy(den_sh.at[:, pl.ds(nbase, SLICE)], red_v)

    @plsc.parallel_loop(0, SLICE // L, unroll=4)
    def _(j):
        acc = red_v[0, pl.ds(j * L, L)]
        for k in range(1, NTILES):
            acc = acc + red_v[k, pl.ds(j * L, L)]
        den_v[pl.ds(j * L, L)] = acc

    pltpu.sync_copy(den_v.at[pl.ds(0, SLICE)], den_out.at[cid, pl.ds(nbase, SLICE)])


def _combine_body(xp_ref, n_ref, d_ref, b_ref, o_ref):
    num = xp_ref[...] + n_ref[0:1, :N] + n_ref[1:2, :N]
    den = 1.0 + d_ref[0:1, :N] + d_ref[1:2, :N] + 1e-16
    o_ref[...] = (num / den + b_ref[0])[0]


def _combine_pallas(xp_pad, num_p, den_p, bias):
    return pl.pallas_call(
        _combine_body,
        out_shape=jax.ShapeDtypeStruct((N,), jnp.float32),
        in_specs=[
            pl.BlockSpec(memory_space=pltpu.VMEM),
            pl.BlockSpec(memory_space=pltpu.VMEM),
            pl.BlockSpec(memory_space=pltpu.VMEM),
            pl.BlockSpec(memory_space=pltpu.SMEM),
        ],
    )(xp_pad, num_p, den_p, bias)


def kernel(F, edge_index, W, att_src, att_dst, bias):
    xp2 = _xp_pallas(F, W)
    num_p, den_p = _edge_kernel(xp2, edge_index,
                                att_src.astype(jnp.float32),
                                att_dst.astype(jnp.float32))
    return _combine_pallas(xp2, num_p, den_p, bias)


# speedup vs baseline: 1.0619x; 1.0378x over previous
import functools

import jax
import jax.numpy as jnp
from jax import lax
from jax.experimental import pallas as pl
from jax.experimental.pallas import tpu as pltpu
from jax.experimental.pallas import tpu_sc as plsc

N = 10000
E = 320000
IN_DIM = 128

L = 16
NCORES = 2
NTILES = 16
NW = NCORES * NTILES
NP = 10240
SLICE = NP // NTILES
EPT = E // NW
ECHUNK = 10240


def _matvec_body(f_ref, w_ref, xp_ref):
    xp_ref[...] = jnp.dot(w_ref[...].T, f_ref[...],
                          preferred_element_type=jnp.float32)


def _xp_pallas(f, w):
    return pl.pallas_call(
        _matvec_body,
        out_shape=jax.ShapeDtypeStruct((1, N), jnp.float32),
    )(f, w)


_sc_mesh = plsc.VectorSubcoreMesh(core_axis_name="c", subcore_axis_name="s")


@functools.partial(
    pl.kernel,
    out_type=[
        jax.ShapeDtypeStruct((NCORES, NP), jnp.float32),
        jax.ShapeDtypeStruct((NCORES, NP), jnp.float32),
    ],
    mesh=_sc_mesh,
    compiler_params=pltpu.CompilerParams(needs_layout_passes=False),
    scratch_types=[
        pltpu.VMEM((N,), jnp.float32),
        pltpu.VMEM((2, ECHUNK), jnp.int32),
        pltpu.VMEM((NP,), jnp.float32),
        pltpu.VMEM((NP,), jnp.float32),
        pltpu.VMEM((1,), jnp.float32),
        pltpu.VMEM((1,), jnp.float32),
        pltpu.VMEM((NTILES, SLICE), jnp.float32),
        pltpu.VMEM((NTILES, SLICE), jnp.float32),
        pltpu.VMEM_SHARED((NTILES, NP), jnp.float32),
        pltpu.VMEM_SHARED((NTILES, NP), jnp.float32),
        pltpu.SemaphoreType.DMA,
        pltpu.SemaphoreType.DMA,
        pltpu.SemaphoreType.DMA,
        pltpu.SemaphoreType.DMA,
    ],
)
def _edge_kernel(xp_hbm, edge_hbm, asrc_hbm, adst_hbm, num_out, den_out,
                 xp_v, ev_v, num_v, den_v, as_v, ad_v, red_v, red2_v,
                 num_sh, den_sh, sem0, sem1, sem2, sem3):
    cid = lax.axis_index("c")
    sid = lax.axis_index("s")
    wid = sid * NCORES + cid

    base = wid * EPT
    abase = jnp.minimum(base - lax.rem(base, 128), E - ECHUNK)
    abase = pl.multiple_of(abase, 128)
    o = base - abase

    cp_e = pltpu.async_copy(edge_hbm.at[:, pl.ds(abase, ECHUNK)], ev_v, sem0)
    cp_x = pltpu.async_copy(xp_hbm.at[0], xp_v, sem1)
    cp_s = pltpu.async_copy(asrc_hbm, as_v, sem2)
    cp_d = pltpu.async_copy(adst_hbm, ad_v, sem3)

    zero = jnp.zeros((L,), jnp.float32)

    @plsc.parallel_loop(0, NP // L, unroll=4)
    def _(j):
        num_v[pl.ds(j * L, L)] = zero
        den_v[pl.ds(j * L, L)] = zero

    cp_e.wait()
    cp_x.wait()
    cp_s.wait()
    cp_d.wait()

    zidx = jnp.zeros((L,), jnp.int32)
    a_s = plsc.load_gather(as_v, [zidx])
    a_d = plsc.load_gather(ad_v, [zidx])
    a_sum = a_s + a_d

    @plsc.parallel_loop(0, EPT // L, unroll=5)
    def _(i):
        sv = ev_v[0, pl.ds(o + i * L, L)]
        dv = ev_v[1, pl.ds(o + i * L, L)]
        xs = plsc.load_gather(xp_v, [sv])
        xd = plsc.load_gather(xp_v, [dv])
        t = a_s * xs + a_d * xd
        alpha = jnp.where(t < 0.0, t * 0.2, t)
        c = a_sum * xd
        c = jnp.where(c < 0.0, c * 0.2, c)
        ex = jnp.exp(alpha - c)
        plsc.addupdate_scatter(den_v, [dv], ex)
        plsc.addupdate_scatter(num_v, [dv], ex * xs)

    cp_pn = pltpu.async_copy(num_v, num_sh.at[sid], sem0)
    cp_pd = pltpu.async_copy(den_v, den_sh.at[sid], sem1)
    cp_pn.wait()
    cp_pd.wait()
    plsc.subcore_barrier()

    nbase = sid * SLICE

    cp_rn = pltpu.async_copy(num_sh.at[:, pl.ds(nbase, SLICE)], red_v, sem0)
    cp_rd = pltpu.async_copy(den_sh.at[:, pl.ds(nbase, SLICE)], red2_v, sem1)
    cp_rn.wait()

    @plsc.parallel_loop(0, SLICE // L, unroll=4)
    def _(j):
        acc = red_v[0, pl.ds(j * L, L)]
        for k in range(1, NTILES):
            acc = acc + red_v[k, pl.ds(j * L, L)]
        num_v[pl.ds(j * L, L)] = acc

    cp_on = pltpu.async_copy(num_v.at[pl.ds(0, SLICE)],
                             num_out.at[cid, pl.ds(nbase, SLICE)], sem2)
    cp_rd.wait()

    @plsc.parallel_loop(0, SLICE // L, unroll=4)
    def _(j):
        acc = red2_v[0, pl.ds(j * L, L)]
        for k in range(1, NTILES):
            acc = acc + red2_v[k, pl.ds(j * L, L)]
        den_v[pl.ds(j * L, L)] = acc

    cp_od = pltpu.async_copy(den_v.at[pl.ds(0, SLICE)],
                             den_out.at[cid, pl.ds(nbase, SLICE)], sem3)
    cp_on.wait()
    cp_od.wait()


def _combine_body(xp_ref, n_ref, d_ref, b_ref, o_ref):
    num = xp_ref[...] + n_ref[0:1, :N] + n_ref[1:2, :N]
    den = 1.0 + d_ref[0:1, :N] + d_ref[1:2, :N] + 1e-16
    o_ref[...] = (num / den + b_ref[0])[0]


def _combine_pallas(xp_pad, num_p, den_p, bias):
    return pl.pallas_call(
        _combine_body,
        out_shape=jax.ShapeDtypeStruct((N,), jnp.float32),
        in_specs=[
            pl.BlockSpec(memory_space=pltpu.VMEM),
            pl.BlockSpec(memory_space=pltpu.VMEM),
            pl.BlockSpec(memory_space=pltpu.VMEM),
            pl.BlockSpec(memory_space=pltpu.SMEM),
        ],
    )(xp_pad, num_p, den_p, bias)


def kernel(F, edge_index, W, att_src, att_dst, bias):
    xp2 = _xp_pallas(F, W)
    num_p, den_p = _edge_kernel(xp2, edge_index,
                                att_src.astype(jnp.float32),
                                att_dst.astype(jnp.float32))
    return _combine_pallas(xp2, num_p, den_p, bias)
